# baseline (speedup 1.0000x reference)
import functools

import jax
import jax.numpy as jnp
from jax import lax
from jax.experimental import pallas as pl
from jax.experimental.pallas import tpu as pltpu
from jax.experimental.pallas import tpu_sc as plsc

N = 10000
NP = 10240
D = 128
DH = 64
E = 160000
NSUB = 16
K = 128
GL = 3
CH = 84
EPT = CH * K
RPT = NP // NSUB
ZR = 128
DEGW = 16
NB = 4


def _sc_agg_body(xlo_hbm, xhi_hbm, src_hbm, dst_hbm, olo_hbm, ohi_hbm,
                 srcb, dstb, rows, zbuf, acc, gsems, ssems):
  c = lax.axis_index("c")
  s = lax.axis_index("s")

  z16 = jnp.zeros((16,), jnp.float32)

  def _zrow(r, _):
    def _zcol(k8, _):
      zbuf[r, pl.ds(k8 * 16, 16)] = z16
      return 0
    lax.fori_loop(0, DH // 16, _zcol, 0)
    return 0
  lax.fori_loop(0, ZR, _zrow, 0)

  row0 = (c * NSUB + s) * CH
  pltpu.sync_copy(src_hbm.at[pl.ds(row0, CH)], srcb)
  pltpu.sync_copy(dst_hbm.at[pl.ds(row0, CH)], dstb)

  for x_hbm, out_hbm in ((xlo_hbm, olo_hbm), (xhi_hbm, ohi_hbm)):
    def _zacc(k, _):
      pltpu.sync_copy(zbuf, acc.at[pl.ds(s * RPT + k * ZR, ZR)])
      return 0
    lax.fori_loop(0, RPT // ZR, _zacc, 0)

    plsc.subcore_barrier()

    def _fire_g(i, grp, sem):
      for b in range(GL):
        pltpu.async_copy(x_hbm.at[srcb.at[i * GL + b]],
                         rows[grp * GL + b], sem)

    def _drain_g(grp, sem):
      for b in range(GL):
        pltpu.make_async_copy(x_hbm.at[srcb.at[0]], rows[grp * GL + b],
                              sem).wait()

    def _fire_s(i, grp, sem):
      for b in range(GL):
        pltpu.async_copy(rows[grp * GL + b], acc.at[dstb.at[i * GL + b]],
                         sem, add=True)

    def _drain_s(grp, sem):
      for b in range(GL):
        pltpu.make_async_copy(rows[grp * GL + b], acc.at[dstb.at[0]],
                              sem).wait()

    NG = CH // GL
    _fire_g(0, 0, gsems[0])

    def _pair(it, _):
      for par in range(2):
        i = 2 * it + par
        grp, ogrp = par, 1 - par
        _drain_g(grp, gsems[par])
        @pl.when(i >= 1)
        def _():
          _drain_s(ogrp, ssems[ogrp])
        @pl.when(i + 1 < NG)
        def _():
          _fire_g(i + 1, ogrp, gsems[ogrp])
        _fire_s(i, grp, ssems[par])
      return 0
    lax.fori_loop(0, NG // 2, _pair, 0)

    _drain_s(1, ssems[1])

    plsc.subcore_barrier()

    def _wout(k, _):
      r = s * RPT + k * ZR
      pltpu.sync_copy(acc.at[pl.ds(r, ZR)], zbuf)
      pltpu.sync_copy(zbuf, out_hbm.at[pl.ds(c * NP + r, ZR)])
      return 0
    lax.fori_loop(0, RPT // ZR, _wout, 0)

    lax.fori_loop(0, ZR, _zrow, 0)


def _make_sc_agg():
  mesh = plsc.VectorSubcoreMesh(core_axis_name="c", subcore_axis_name="s")
  out_type = [jax.ShapeDtypeStruct((2 * NP, DH), jnp.float32),
              jax.ShapeDtypeStruct((2 * NP, DH), jnp.float32)]
  scratch = [
      pltpu.VMEM((CH, K), jnp.int32),
      pltpu.VMEM((CH, K), jnp.int32),
      [pltpu.VMEM((K, DH), jnp.float32) for _ in range(2 * GL)],
      pltpu.VMEM((ZR, DH), jnp.float32),
      pltpu.VMEM_SHARED((NP, DH), jnp.float32),
      [pltpu.SemaphoreType.DMA for _ in range(2)],
      [pltpu.SemaphoreType.DMA for _ in range(2)],
  ]
  return pl.kernel(_sc_agg_body,
                   out_type=out_type, mesh=mesh, scratch_types=scratch,
                   compiler_params=pltpu.CompilerParams(
                       use_tc_tiling_on_sc=False))


def _sc_deg_body(dst_hbm, deg_hbm, dstb, ones, degbuf, dega, sems):
  c = lax.axis_index("c")
  s = lax.axis_index("s")

  z16 = jnp.zeros((16,), jnp.float32)
  o16 = jnp.ones((16,), jnp.float32)

  def _irow(r, _):
    degbuf[r, :] = z16
    return 0
  lax.fori_loop(0, ZR, _irow, 0)
  def _orow(r, _):
    ones[r, :] = o16
    return 0
  lax.fori_loop(0, K, _orow, 0)

  def _zacc(k, _):
    pltpu.sync_copy(degbuf, dega.at[pl.ds(s * RPT + k * ZR, ZR)])
    return 0
  lax.fori_loop(0, RPT // ZR, _zacc, 0)

  plsc.subcore_barrier()

  row0 = (c * NSUB + s) * CH
  pltpu.sync_copy(dst_hbm.at[pl.ds(row0, CH)], dstb)

  QD = 4
  def _dchunk(j, _):
    @pl.when(j >= QD)
    def _():
      pltpu.make_async_copy(ones, dega.at[dstb.at[j]], sems[0]).wait()
    pltpu.async_copy(ones, dega.at[dstb.at[j]], sems[0], add=True)
    return 0
  lax.fori_loop(0, CH, _dchunk, 0)
  def _ddrain(j, _):
    pltpu.make_async_copy(ones, dega.at[dstb.at[0]], sems[0]).wait()
    return 0
  lax.fori_loop(0, QD, _ddrain, 0)

  plsc.subcore_barrier()

  def _wout(k, _):
    r = s * RPT + k * ZR
    pltpu.sync_copy(dega.at[pl.ds(r, ZR)], degbuf)
    pltpu.sync_copy(degbuf, deg_hbm.at[pl.ds(c * NP + r, ZR)])
    return 0
  lax.fori_loop(0, RPT // ZR, _wout, 0)


def _make_sc_deg():
  mesh = plsc.VectorSubcoreMesh(core_axis_name="c", subcore_axis_name="s")
  out_type = jax.ShapeDtypeStruct((2 * NP, DEGW), jnp.float32)
  scratch = [
      pltpu.VMEM((CH, K), jnp.int32),
      pltpu.VMEM((K, DEGW), jnp.float32),
      pltpu.VMEM((ZR, DEGW), jnp.float32),
      pltpu.VMEM_SHARED((NP, DEGW), jnp.float32),
      [pltpu.SemaphoreType.DMA for _ in range(1)],
  ]
  return pl.kernel(_sc_deg_body,
                   out_type=out_type, mesh=mesh, scratch_types=scratch,
                   compiler_params=pltpu.CompilerParams(
                       use_tc_tiling_on_sc=False))


_sc_agg = _make_sc_agg()
_sc_deg = _make_sc_deg()

BR = 1280


def _tc_dense_body(relu, split_out, xlo_ref, xhi_ref, alo_ref, ahi_ref,
                   deg_ref, basis_ref, comp_ref, root_ref, bias_ref,
                   *o_refs):
  g = pl.program_id(0)
  wu = jnp.zeros((D, D), jnp.float32)
  wi = jnp.zeros((D, D), jnp.float32)
  for b in range(NB):
    wu = wu + comp_ref[1, b] * basis_ref[b]
    wi = wi + comp_ref[0, b] * basis_ref[b]
  w = jnp.where(g < (NP // BR), wu, wi)
  dinv = 1.0 / jnp.maximum(deg_ref[:, 0:1], 1.0)
  root = root_ref[...]
  h = (jnp.dot(xlo_ref[...], root[:DH], preferred_element_type=jnp.float32)
       + jnp.dot(xhi_ref[...], root[DH:], preferred_element_type=jnp.float32)
       + jnp.dot(alo_ref[...] * dinv, w[:DH],
                 preferred_element_type=jnp.float32)
       + jnp.dot(ahi_ref[...] * dinv, w[DH:],
                 preferred_element_type=jnp.float32)
       + bias_ref[...])
  if relu:
    h = jnp.maximum(h, 0.0)
  if split_out:
    o_refs[0][...] = h[:, :DH]
    o_refs[1][...] = h[:, DH:]
  else:
    o_refs[0][...] = h


def _make_tc_dense(relu, split_out):
  half = pl.BlockSpec((BR, DH), lambda g: (g, 0))
  full = pl.BlockSpec((BR, D), lambda g: (g, 0))
  if split_out:
    out_specs = [half, half]
    out_shape = [jax.ShapeDtypeStruct((2 * NP, DH), jnp.float32),
                 jax.ShapeDtypeStruct((2 * NP, DH), jnp.float32)]
  else:
    out_specs = [full]
    out_shape = [jax.ShapeDtypeStruct((2 * NP, D), jnp.float32)]

  return pl.pallas_call(
      functools.partial(_tc_dense_body, relu, split_out),
      grid=(2 * NP // BR,),
      in_specs=[
          half, half, half, half,
          pl.BlockSpec((BR, DEGW), lambda g: (g, 0)),
          pl.BlockSpec((NB, D, D), lambda g: (0, 0, 0)),
          pl.BlockSpec(memory_space=pltpu.SMEM),
          pl.BlockSpec((D, D), lambda g: (0, 0)),
          pl.BlockSpec((1, D), lambda g: (0, 0)),
      ],
      out_specs=out_specs,
      out_shape=out_shape,
  )


_tc_dense_relu = _make_tc_dense(True, True)
_tc_dense_lin = _make_tc_dense(False, False)


def _prep_edges(ei, src_off):
  src = (ei[0].astype(jnp.int32) + src_off).reshape(NSUB, E // NSUB)
  dst = ei[1].astype(jnp.int32).reshape(NSUB, E // NSUB)
  pad = EPT - E // NSUB
  src = jnp.pad(src, ((0, 0), (0, pad)), constant_values=0)
  dst = jnp.pad(dst, ((0, 0), (0, pad)), constant_values=N)
  return src, dst


def _pad_half(xu, xi, col):
  zpad = jnp.zeros((NP - N, DH), jnp.float32)
  return jnp.concatenate([xu[:, col:col + DH], zpad,
                          xi[:, col:col + DH], zpad], axis=0)


def kernel(x_user, x_item, edge_index_user_item, edge_index_item_user,
           basis0, comp0, root0, bias0, basis1, comp1, root1, bias1):
  s0, d0 = _prep_edges(edge_index_item_user, NP)
  s1, d1 = _prep_edges(edge_index_user_item, 0)
  src2 = jnp.stack([s0, s1]).reshape(2 * NSUB * CH, K)
  dst2 = jnp.stack([d0, d1]).reshape(2 * NSUB * CH, K)

  x0lo = _pad_half(x_user, x_item, 0)
  x0hi = _pad_half(x_user, x_item, DH)

  deg = _sc_deg(dst2)
  src2d = src2 + (0.0 * deg[0, 0]).astype(jnp.int32)
  a0lo, a0hi = _sc_agg(x0lo, x0hi, src2d, dst2)
  h1lo, h1hi = _tc_dense_relu(x0lo, x0hi, a0lo, a0hi, deg, basis0, comp0,
                              root0, bias0.reshape(1, D))
  a1lo, a1hi = _sc_agg(h1lo, h1hi, src2d, dst2)
  (h2,) = _tc_dense_lin(h1lo, h1hi, a1lo, a1hi, deg, basis1, comp1,
                        root1, bias1.reshape(1, D))
  return (h2[:N], h2[NP:NP + N])


# speedup vs baseline: 2.2336x; 2.2336x over previous
import functools

import jax
import jax.numpy as jnp
from jax import lax
from jax.experimental import pallas as pl
from jax.experimental.pallas import tpu as pltpu
from jax.experimental.pallas import tpu_sc as plsc

N = 10000
NP = 10240
D = 128
DH = 64
E = 160000
NSUB = 16
K = 128
PD = 4
CH = 80
EPT = CH * K
RPT = NP // NSUB
ZR = 128
DEGW = 16
NB = 4


def _sc_agg_body(x_hbm, src_hbm, dst_hbm, out_hbm,
                 srcb, dstb, rows, zbuf, acc, sems):
  c = lax.axis_index("c")
  s = lax.axis_index("s")

  z16 = jnp.zeros((16,), jnp.float32)

  def _zrow(r, _):
    def _zcol(k8, _):
      zbuf[r, pl.ds(k8 * 16, 16)] = z16
      return 0
    lax.fori_loop(0, DH // 16, _zcol, 0)
    return 0
  lax.fori_loop(0, ZR, _zrow, 0)

  def _zacc(k, _):
    pltpu.sync_copy(zbuf, acc.at[pl.ds(s * RPT + k * ZR, ZR)])
    return 0
  lax.fori_loop(0, RPT // ZR, _zacc, 0)

  plsc.subcore_barrier()

  row0 = (c * NSUB + s) * CH
  pltpu.sync_copy(src_hbm.at[pl.ds(row0, CH)], srcb)
  pltpu.sync_copy(dst_hbm.at[pl.ds(row0, CH)], dstb)

  def _fire(j, b):
    pltpu.async_copy(x_hbm.at[srcb.at[j]], rows[b], sems[b])

  for b in range(PD):
    _fire(b, b)

  def _group(g, _):
    for b in range(PD):
      j = g * PD + b
      pltpu.make_async_copy(x_hbm.at[srcb.at[j]], rows[b], sems[b]).wait()
      pltpu.sync_copy(rows[b], acc.at[dstb.at[j]], add=True)
      @pl.when(j + PD < CH)
      def _():
        _fire(j + PD, b)
    return 0
  lax.fori_loop(0, CH // PD, _group, 0)

  plsc.subcore_barrier()

  def _wout(k, _):
    r = s * RPT + k * ZR
    pltpu.sync_copy(acc.at[pl.ds(r, ZR)], zbuf)
    pltpu.sync_copy(zbuf, out_hbm.at[pl.ds(c * NP + r, ZR)])
    return 0
  lax.fori_loop(0, RPT // ZR, _wout, 0)


def _make_sc_agg():
  mesh = plsc.VectorSubcoreMesh(core_axis_name="c", subcore_axis_name="s")
  out_type = jax.ShapeDtypeStruct((2 * NP, DH), jnp.float32)
  scratch = [
      pltpu.VMEM((CH, K), jnp.int32),
      pltpu.VMEM((CH, K), jnp.int32),
      [pltpu.VMEM((K, DH), jnp.float32) for _ in range(PD)],
      pltpu.VMEM((ZR, DH), jnp.float32),
      pltpu.VMEM_SHARED((NP, DH), jnp.float32),
      [pltpu.SemaphoreType.DMA for _ in range(PD)],
  ]
  return pl.kernel(_sc_agg_body,
                   out_type=out_type, mesh=mesh, scratch_types=scratch,
                   compiler_params=pltpu.CompilerParams(
                       use_tc_tiling_on_sc=False))


def _sc_deg_body(dst_hbm, deg_hbm, dstb, ones, degbuf, dega, sems):
  c = lax.axis_index("c")
  s = lax.axis_index("s")

  z16 = jnp.zeros((16,), jnp.float32)
  o16 = jnp.ones((16,), jnp.float32)

  def _irow(r, _):
    degbuf[r, :] = z16
    return 0
  lax.fori_loop(0, ZR, _irow, 0)
  def _orow(r, _):
    ones[r, :] = o16
    return 0
  lax.fori_loop(0, K, _orow, 0)

  def _zacc(k, _):
    pltpu.sync_copy(degbuf, dega.at[pl.ds(s * RPT + k * ZR, ZR)])
    return 0
  lax.fori_loop(0, RPT // ZR, _zacc, 0)

  plsc.subcore_barrier()

  row0 = (c * NSUB + s) * CH
  pltpu.sync_copy(dst_hbm.at[pl.ds(row0, CH)], dstb)

  QD = 4
  def _dchunk(j, _):
    @pl.when(j >= QD)
    def _():
      pltpu.make_async_copy(ones, dega.at[dstb.at[j]], sems[0]).wait()
    pltpu.async_copy(ones, dega.at[dstb.at[j]], sems[0], add=True)
    return 0
  lax.fori_loop(0, CH, _dchunk, 0)
  def _ddrain(j, _):
    pltpu.make_async_copy(ones, dega.at[dstb.at[0]], sems[0]).wait()
    return 0
  lax.fori_loop(0, QD, _ddrain, 0)

  plsc.subcore_barrier()

  def _wout(k, _):
    r = s * RPT + k * ZR
    pltpu.sync_copy(dega.at[pl.ds(r, ZR)], degbuf)
    pltpu.sync_copy(degbuf, deg_hbm.at[pl.ds(c * NP + r, ZR)])
    return 0
  lax.fori_loop(0, RPT // ZR, _wout, 0)


def _make_sc_deg():
  mesh = plsc.VectorSubcoreMesh(core_axis_name="c", subcore_axis_name="s")
  out_type = jax.ShapeDtypeStruct((2 * NP, DEGW), jnp.float32)
  scratch = [
      pltpu.VMEM((CH, K), jnp.int32),
      pltpu.VMEM((K, DEGW), jnp.float32),
      pltpu.VMEM((ZR, DEGW), jnp.float32),
      pltpu.VMEM_SHARED((NP, DEGW), jnp.float32),
      [pltpu.SemaphoreType.DMA for _ in range(1)],
  ]
  return pl.kernel(_sc_deg_body,
                   out_type=out_type, mesh=mesh, scratch_types=scratch,
                   compiler_params=pltpu.CompilerParams(
                       use_tc_tiling_on_sc=False))


_sc_agg = _make_sc_agg()
_sc_deg = _make_sc_deg()

BR = 1280


def _tc_dense_body(relu, split_out, xlo_ref, xhi_ref, alo_ref, ahi_ref,
                   deg_ref, basis_ref, comp_ref, root_ref, bias_ref,
                   *o_refs):
  g = pl.program_id(0)
  wu = jnp.zeros((D, D), jnp.float32)
  wi = jnp.zeros((D, D), jnp.float32)
  for b in range(NB):
    wu = wu + comp_ref[1, b] * basis_ref[b]
    wi = wi + comp_ref[0, b] * basis_ref[b]
  w = jnp.where(g < (NP // BR), wu, wi)
  dinv = 1.0 / jnp.maximum(deg_ref[:, 0:1], 1.0)
  root = root_ref[...]
  h = (jnp.dot(xlo_ref[...], root[:DH], preferred_element_type=jnp.float32)
       + jnp.dot(xhi_ref[...], root[DH:], preferred_element_type=jnp.float32)
       + jnp.dot(alo_ref[...] * dinv, w[:DH],
                 preferred_element_type=jnp.float32)
       + jnp.dot(ahi_ref[...] * dinv, w[DH:],
                 preferred_element_type=jnp.float32)
       + bias_ref[...])
  if relu:
    h = jnp.maximum(h, 0.0)
  if split_out:
    o_refs[0][...] = h[:, :DH]
    o_refs[1][...] = h[:, DH:]
  else:
    o_refs[0][...] = h


def _make_tc_dense(relu, split_out):
  half = pl.BlockSpec((BR, DH), lambda g: (g, 0))
  full = pl.BlockSpec((BR, D), lambda g: (g, 0))
  if split_out:
    out_specs = [half, half]
    out_shape = [jax.ShapeDtypeStruct((2 * NP, DH), jnp.float32),
                 jax.ShapeDtypeStruct((2 * NP, DH), jnp.float32)]
  else:
    out_specs = [full]
    out_shape = [jax.ShapeDtypeStruct((2 * NP, D), jnp.float32)]

  return pl.pallas_call(
      functools.partial(_tc_dense_body, relu, split_out),
      grid=(2 * NP // BR,),
      in_specs=[
          half, half, half, half,
          pl.BlockSpec((BR, DEGW), lambda g: (g, 0)),
          pl.BlockSpec((NB, D, D), lambda g: (0, 0, 0)),
          pl.BlockSpec(memory_space=pltpu.SMEM),
          pl.BlockSpec((D, D), lambda g: (0, 0)),
          pl.BlockSpec((1, D), lambda g: (0, 0)),
      ],
      out_specs=out_specs,
      out_shape=out_shape,
  )


_tc_dense_relu = _make_tc_dense(True, True)
_tc_dense_lin = _make_tc_dense(False, False)


def _prep_edges(ei, src_off):
  src = (ei[0].astype(jnp.int32) + src_off).reshape(NSUB, E // NSUB)
  dst = ei[1].astype(jnp.int32).reshape(NSUB, E // NSUB)
  pad = EPT - E // NSUB
  src = jnp.pad(src, ((0, 0), (0, pad)), constant_values=0)
  dst = jnp.pad(dst, ((0, 0), (0, pad)), constant_values=N)
  return src, dst


def _pad_half(xu, xi, col):
  zpad = jnp.zeros((NP - N, DH), jnp.float32)
  return jnp.concatenate([xu[:, col:col + DH], zpad,
                          xi[:, col:col + DH], zpad], axis=0)


def kernel(x_user, x_item, edge_index_user_item, edge_index_item_user,
           basis0, comp0, root0, bias0, basis1, comp1, root1, bias1):
  s0, d0 = _prep_edges(edge_index_item_user, NP)
  s1, d1 = _prep_edges(edge_index_user_item, 0)
  src2 = jnp.stack([s0, s1]).reshape(2 * NSUB * CH, K)
  dst2 = jnp.stack([d0, d1]).reshape(2 * NSUB * CH, K)

  x0lo = _pad_half(x_user, x_item, 0)
  x0hi = _pad_half(x_user, x_item, DH)

  deg = _sc_deg(dst2)
  src2d = src2 + (0.0 * deg[0, 0]).astype(jnp.int32)
  a0lo = _sc_agg(x0lo, src2d, dst2)
  a0hi = _sc_agg(x0hi + 0.0 * a0lo[0, 0], src2d, dst2)
  h1lo, h1hi = _tc_dense_relu(x0lo, x0hi, a0lo, a0hi, deg, basis0, comp0,
                              root0, bias0.reshape(1, D))
  a1lo = _sc_agg(h1lo, src2d, dst2)
  a1hi = _sc_agg(h1hi + 0.0 * a1lo[0, 0], src2d, dst2)
  (h2,) = _tc_dense_lin(h1lo, h1hi, a1lo, a1hi, deg, basis1, comp1,
                        root1, bias1.reshape(1, D))
  return (h2[:N], h2[NP:NP + N])


# speedup vs baseline: 2.3526x; 1.0533x over previous
import functools

import jax
import jax.numpy as jnp
from jax import lax
from jax.experimental import pallas as pl
from jax.experimental.pallas import tpu as pltpu
from jax.experimental.pallas import tpu_sc as plsc

N = 10000
NP = 10240
D = 128
DH = 64
E = 160000
NSUB = 16
K = 128
PD = 4
CH = 80
EPT = CH * K
RPT = NP // NSUB
ZR = 128
DEGW = 16
NB = 4


def _sc_agg_body(with_deg, x_hbm, src_hbm, dst_hbm, out_hbm, deg_hbm,
                 srcb, dstb, rows, ones, zbuf, degbuf, acc, dega, sems):
  c = lax.axis_index("c")
  s = lax.axis_index("s")

  z16 = jnp.zeros((16,), jnp.float32)

  def _zrow(r, _):
    def _zcol(k8, _):
      zbuf[r, pl.ds(k8 * 16, 16)] = z16
      return 0
    lax.fori_loop(0, DH // 16, _zcol, 0)
    if with_deg:
      degbuf[r, :] = z16
    return 0
  lax.fori_loop(0, ZR, _zrow, 0)

  if with_deg:
    o16 = jnp.ones((16,), jnp.float32)
    def _orow(r, _):
      ones[r, :] = o16
      return 0
    lax.fori_loop(0, K, _orow, 0)

  def _zacc(k, _):
    pltpu.sync_copy(zbuf, acc.at[pl.ds(s * RPT + k * ZR, ZR)])
    if with_deg:
      pltpu.sync_copy(degbuf, dega.at[pl.ds(s * RPT + k * ZR, ZR)])
    return 0
  lax.fori_loop(0, RPT // ZR, _zacc, 0)

  plsc.subcore_barrier()

  row0 = (c * NSUB + s) * CH
  pltpu.sync_copy(src_hbm.at[pl.ds(row0, CH)], srcb)
  pltpu.sync_copy(dst_hbm.at[pl.ds(row0, CH)], dstb)

  def _fire(j, b):
    pltpu.async_copy(x_hbm.at[srcb.at[j]], rows[b], sems[b])

  for b in range(PD):
    _fire(b, b)

  def _group(g, _):
    for b in range(PD):
      j = g * PD + b
      pltpu.make_async_copy(x_hbm.at[srcb.at[j]], rows[b], sems[b]).wait()
      pltpu.sync_copy(rows[b], acc.at[dstb.at[j]], add=True)
      if with_deg:
        pltpu.sync_copy(ones, dega.at[dstb.at[j]], add=True)
      @pl.when(j + PD < CH)
      def _():
        _fire(j + PD, b)
    return 0
  lax.fori_loop(0, CH // PD, _group, 0)

  plsc.subcore_barrier()

  def _wout(k, _):
    r = s * RPT + k * ZR
    pltpu.sync_copy(acc.at[pl.ds(r, ZR)], zbuf)
    pltpu.sync_copy(zbuf, out_hbm.at[pl.ds(c * NP + r, ZR)])
    if with_deg:
      pltpu.sync_copy(dega.at[pl.ds(r, ZR)], degbuf)
      pltpu.sync_copy(degbuf, deg_hbm.at[pl.ds(c * NP + r, ZR)])
    return 0
  lax.fori_loop(0, RPT // ZR, _wout, 0)


def _make_sc_agg(with_deg):
  mesh = plsc.VectorSubcoreMesh(core_axis_name="c", subcore_axis_name="s")
  out_type = [jax.ShapeDtypeStruct((2 * NP, DH), jnp.float32),
              jax.ShapeDtypeStruct((2 * NP, DEGW), jnp.float32)]
  scratch = [
      pltpu.VMEM((CH, K), jnp.int32),
      pltpu.VMEM((CH, K), jnp.int32),
      [pltpu.VMEM((K, DH), jnp.float32) for _ in range(PD)],
      pltpu.VMEM((K, DEGW), jnp.float32),
      pltpu.VMEM((ZR, DH), jnp.float32),
      pltpu.VMEM((ZR, DEGW), jnp.float32),
      pltpu.VMEM_SHARED((NP, DH), jnp.float32),
      pltpu.VMEM_SHARED((NP, DEGW), jnp.float32),
      [pltpu.SemaphoreType.DMA for _ in range(PD)],
  ]
  return pl.kernel(functools.partial(_sc_agg_body, with_deg),
                   out_type=out_type, mesh=mesh, scratch_types=scratch,
                   compiler_params=pltpu.CompilerParams(
                       use_tc_tiling_on_sc=False))


_sc_agg_deg = _make_sc_agg(True)
_sc_agg_nd = _make_sc_agg(False)

BR = 1280


def _tc_dense_body(relu, split_out, xlo_ref, xhi_ref, alo_ref, ahi_ref,
                   deg_ref, basis_ref, comp_ref, root_ref, bias_ref,
                   *o_refs):
  g = pl.program_id(0)
  wu = jnp.zeros((D, D), jnp.float32)
  wi = jnp.zeros((D, D), jnp.float32)
  for b in range(NB):
    wu = wu + comp_ref[1, b] * basis_ref[b]
    wi = wi + comp_ref[0, b] * basis_ref[b]
  w = jnp.where(g < (NP // BR), wu, wi)
  dinv = 1.0 / jnp.maximum(deg_ref[:, 0:1], 1.0)
  root = root_ref[...]
  h = (jnp.dot(xlo_ref[...], root[:DH], preferred_element_type=jnp.float32)
       + jnp.dot(xhi_ref[...], root[DH:], preferred_element_type=jnp.float32)
       + jnp.dot(alo_ref[...] * dinv, w[:DH],
                 preferred_element_type=jnp.float32)
       + jnp.dot(ahi_ref[...] * dinv, w[DH:],
                 preferred_element_type=jnp.float32)
       + bias_ref[...])
  if relu:
    h = jnp.maximum(h, 0.0)
  if split_out:
    o_refs[0][...] = h[:, :DH]
    o_refs[1][...] = h[:, DH:]
  else:
    o_refs[0][...] = h


def _make_tc_dense(relu, split_out):
  half = pl.BlockSpec((BR, DH), lambda g: (g, 0))
  full = pl.BlockSpec((BR, D), lambda g: (g, 0))
  if split_out:
    out_specs = [half, half]
    out_shape = [jax.ShapeDtypeStruct((2 * NP, DH), jnp.float32),
                 jax.ShapeDtypeStruct((2 * NP, DH), jnp.float32)]
  else:
    out_specs = [full]
    out_shape = [jax.ShapeDtypeStruct((2 * NP, D), jnp.float32)]

  return pl.pallas_call(
      functools.partial(_tc_dense_body, relu, split_out),
      grid=(2 * NP // BR,),
      in_specs=[
          half, half, half, half,
          pl.BlockSpec((BR, DEGW), lambda g: (g, 0)),
          pl.BlockSpec((NB, D, D), lambda g: (0, 0, 0)),
          pl.BlockSpec(memory_space=pltpu.SMEM),
          pl.BlockSpec((D, D), lambda g: (0, 0)),
          pl.BlockSpec((1, D), lambda g: (0, 0)),
      ],
      out_specs=out_specs,
      out_shape=out_shape,
  )


_tc_dense_relu = _make_tc_dense(True, True)
_tc_dense_lin = _make_tc_dense(False, False)


def _prep_edges(ei, src_off):
  src = (ei[0].astype(jnp.int32) + src_off).reshape(NSUB, E // NSUB)
  dst = ei[1].astype(jnp.int32).reshape(NSUB, E // NSUB)
  pad = EPT - E // NSUB
  src = jnp.pad(src, ((0, 0), (0, pad)), constant_values=0)
  dst = jnp.pad(dst, ((0, 0), (0, pad)), constant_values=N)
  return src, dst


def _pad_half(xu, xi, col):
  zpad = jnp.zeros((NP - N, DH), jnp.float32)
  return jnp.concatenate([xu[:, col:col + DH], zpad,
                          xi[:, col:col + DH], zpad], axis=0)


def kernel(x_user, x_item, edge_index_user_item, edge_index_item_user,
           basis0, comp0, root0, bias0, basis1, comp1, root1, bias1):
  s0, d0 = _prep_edges(edge_index_item_user, NP)
  s1, d1 = _prep_edges(edge_index_user_item, 0)
  src2 = jnp.stack([s0, s1]).reshape(2 * NSUB * CH, K)
  dst2 = jnp.stack([d0, d1]).reshape(2 * NSUB * CH, K)

  x0lo = _pad_half(x_user, x_item, 0)
  x0hi = _pad_half(x_user, x_item, DH)

  a0lo, deg = _sc_agg_deg(x0lo, src2, dst2)
  a0hi, _ = _sc_agg_nd(x0hi, src2, dst2)
  h1lo, h1hi = _tc_dense_relu(x0lo, x0hi, a0lo, a0hi, deg, basis0, comp0,
                              root0, bias0.reshape(1, D))
  a1lo, _ = _sc_agg_nd(h1lo, src2, dst2)
  a1hi, _ = _sc_agg_nd(h1hi, src2, dst2)
  (h2,) = _tc_dense_lin(h1lo, h1hi, a1lo, a1hi, deg, basis1, comp1,
                        root1, bias1.reshape(1, D))
  return (h2[:N], h2[NP:NP + N])
